# baseline (speedup 1.0000x reference)
import jax
import jax.numpy as jnp
from jax.experimental import pallas as pl
from jax.experimental.pallas import tpu as pltpu

B = 32
KS = 5

_T2_OFF = 544
_T3_OFF = 544
_WH_OFF = 1824
_ROWS = 3104


def _pool(u, h2, n, bias):
    m = jnp.max(u.reshape(h2, 2, B, n), axis=1)
    m = jnp.maximum(m[..., :n // 2], m[..., n // 2:]) + bias.reshape(1, 1, n // 2)
    return m.astype(jnp.bfloat16)


def _net_kernel(x3_ref, w_ref, b_ref, o_ref):
    f32 = jnp.float32
    pad_h = ((2, 2), (0, 0), (0, 0))

    u1 = jnp.dot(x3_ref[...].reshape(32 * B, 540), w_ref[0:540, :],
                 preferred_element_type=f32)
    y1 = jnp.pad(_pool(u1, 16, 1024, b_ref[0:1, :]), pad_h)

    u2 = jnp.dot(y1[0:16].reshape(16 * B, 512),
                 w_ref[_T2_OFF:_T2_OFF + 512, 0:512],
                 preferred_element_type=f32)
    for di in range(1, KS):
        u2 = u2 + jnp.dot(
            y1[di:di + 16].reshape(16 * B, 512),
            w_ref[_T2_OFF + di * 512:_T2_OFF + (di + 1) * 512, 0:512],
            preferred_element_type=f32)
    y2 = jnp.pad(_pool(u2, 8, 512, b_ref[1:2, 0:256]), pad_h)

    u3 = jnp.dot(y2[0:8].reshape(8 * B, 256),
                 w_ref[_T3_OFF:_T3_OFF + 256, 512:1024],
                 preferred_element_type=f32)
    for di in range(1, KS):
        u3 = u3 + jnp.dot(
            y2[di:di + 8].reshape(8 * B, 256),
            w_ref[_T3_OFF + di * 256:_T3_OFF + (di + 1) * 256, 512:1024],
            preferred_element_type=f32)
    y3 = _pool(u3, 4, 512, b_ref[2:3, 0:256])

    out = jnp.zeros((B, 128), f32) + b_ref[3:4, 0:128]
    for h in range(4):
        out = out + jnp.dot(
            y3[h], w_ref[_WH_OFF + h * 256:_WH_OFF + (h + 1) * 256, 512:640],
            preferred_element_type=f32)
    o_ref[...] = out


def _toeplitz(w_dxio, e_dj_list, w_in, c_in, w_out, c_out):
    t = sum(jnp.einsum("xw,dio->dxiwo", e_dj_list[dj], w_dxio[:, dj])
            for dj in range(KS))
    k = KS * w_in * c_in
    t = t.reshape(k, w_out // 2, 2, c_out)
    t = jnp.transpose(t, (0, 2, 1, 3))
    return t.reshape(k, w_out * c_out)


def kernel(c1_w, c1_b, c2_w, c2_b, c3_w, c3_b, head_w, head_b, x_nchw):
    N = x_nchw.shape[0]
    bf16 = jnp.bfloat16
    ar = jnp.arange

    w1 = c1_w[:, :, :32].reshape(KS, KS, 3, 32)
    e1 = [(ar(36)[:, None] == ar(32)[None, :] + dj).astype(jnp.float32)
          for dj in range(KS)]
    t1 = _toeplitz(w1, e1, 36, 3, 32, 32)

    w2 = c2_w[:, :32, :32].reshape(KS, KS, 32, 32)
    e2 = [(ar(16)[:, None] == ar(16)[None, :] + dj - 2).astype(jnp.float32)
          for dj in range(KS)]
    t2 = _toeplitz(w2, e2, 16, 32, 16, 32)

    w3 = c3_w[:, :32, :64].reshape(KS, KS, 32, 64)
    e3 = [(ar(8)[:, None] == ar(8)[None, :] + dj - 2).astype(jnp.float32)
          for dj in range(KS)]
    t3 = _toeplitz(w3, e3, 8, 32, 8, 64)

    wh = head_w.reshape(4, 4, 128, 128)[:, :, :64, :].reshape(1024, 128)

    atlas = jnp.zeros((_ROWS, 1024), jnp.float32)
    atlas = atlas.at[0:540, :].set(t1)
    atlas = atlas.at[_T2_OFF:_T2_OFF + 2560, 0:512].set(t2)
    atlas = atlas.at[_T3_OFF:_T3_OFF + 1280, 512:1024].set(t3)
    atlas = atlas.at[_WH_OFF:_WH_OFF + 1024, 512:640].set(wh)

    biases = jnp.zeros((4, 512), jnp.float32)
    biases = biases.at[0, :].set(jnp.tile(c1_b[0, :32], 16))
    biases = biases.at[1, 0:256].set(jnp.tile(c2_b[0, :32], 8))
    biases = biases.at[2, 0:256].set(jnp.tile(c3_b[0, :64], 4))
    biases = biases.at[3, 0:128].set(head_b)

    x = jnp.transpose(x_nchw, (2, 0, 3, 1))
    x = jnp.pad(x, ((2, 2), (0, 0), (2, 2), (0, 0)))
    x = x.reshape(36, N, 108).astype(bf16)
    x3 = jnp.concatenate([x[i:i + 32] for i in range(KS)], axis=-1)

    out = pl.pallas_call(
        _net_kernel,
        out_shape=jax.ShapeDtypeStruct((N, 128), jnp.float32),
        grid=(N // B,),
        in_specs=[
            pl.BlockSpec((32, B, 540), lambda n: (0, n, 0)),
            pl.BlockSpec((_ROWS, 1024), lambda n: (0, 0)),
            pl.BlockSpec((4, 512), lambda n: (0, 0)),
        ],
        out_specs=pl.BlockSpec((B, 128), lambda n: (n, 0)),
        compiler_params=pltpu.CompilerParams(
            dimension_semantics=("parallel",)),
    )(x3, atlas.astype(bf16), biases)
    return out[:, :10]
